# Initial kernel scaffold; baseline (speedup 1.0000x reference)
import jax
import jax.numpy as jnp
from jax.experimental import pallas as pl


def kernel(frames):
    raise NotImplementedError("write your pallas kernel here")


# speedup vs baseline: 1.2384x; 1.2384x over previous
import jax
import jax.numpy as jnp
from jax.experimental import pallas as pl
from jax.experimental.pallas import tpu as pltpu


def _gather_body(idx_ref, src_ref, out_ref):
    del idx_ref
    out_ref[...] = src_ref[...]


def kernel(frames):
    C, T, H, W = frames.shape
    alpha = 4
    n = T // alpha
    idx = jnp.linspace(0.0, float(T - 1), n).astype(jnp.int32)
    flat = frames.reshape(C * T, H, W)
    row_idx = (jnp.arange(C, dtype=jnp.int32)[:, None] * T + idx[None, :]).reshape(-1)
    slow_flat = pl.pallas_call(
        _gather_body,
        grid_spec=pltpu.PrefetchScalarGridSpec(
            num_scalar_prefetch=1,
            grid=(C * n,),
            in_specs=[pl.BlockSpec((1, H, W), lambda i, idx_ref: (idx_ref[i], 0, 0))],
            out_specs=pl.BlockSpec((1, H, W), lambda i, idx_ref: (i, 0, 0)),
        ),
        out_shape=jax.ShapeDtypeStruct((C * n, H, W), jnp.float32),
    )(row_idx, flat)
    slow = slow_flat.reshape(C, n, H, W)
    return (slow, frames)
